# baseline (speedup 1.0000x reference)
import functools

import jax
import jax.numpy as jnp
from jax import lax
from jax.experimental import pallas as pl
from jax.experimental.pallas import tpu as pltpu
from jax.experimental.pallas import tpu_sc as plsc

NC = 2
NS = 16
NW = NC * NS
N_NEG = 5
CHUNK = 64


def _sc_scores(b_per_w, n_chunk, d):
    mesh = plsc.VectorSubcoreMesh(core_axis_name="c", subcore_axis_name="s")
    batch = b_per_w * NW
    w2 = 2 * d

    @functools.partial(
        pl.kernel,
        mesh=mesh,
        out_type=jax.ShapeDtypeStruct((8 * batch,), jnp.float32),
        compiler_params=pltpu.CompilerParams(needs_layout_passes=False,
                                             use_tc_tiling_on_sc=True),
        scratch_types=[
            pltpu.VMEM((b_per_w,), jnp.int32),
            pltpu.VMEM((b_per_w,), jnp.int32),
            pltpu.VMEM((b_per_w,), jnp.int32),
            pltpu.VMEM((b_per_w,), jnp.int32),
            pltpu.VMEM((N_NEG * b_per_w,), jnp.int32),
            pltpu.VMEM((N_NEG * b_per_w,), jnp.int32),
            pltpu.VMEM((CHUNK, w2), jnp.float32),
            pltpu.VMEM((CHUNK, w2), jnp.float32),
            pltpu.VMEM((N_NEG * CHUNK, w2), jnp.float32),
            pltpu.VMEM((6, b_per_w), jnp.float32),
            pltpu.SemaphoreType.DMA,
        ],
    )
    def k(pu_h, pu_p, pv_h, pv_p, ng_h, ng_p, u2, v2, out,
          idx_u, par_u, idx_v, par_v, idx_n, par_n,
          u_buf, v_buf, n_buf, acc_buf, sem):
        wid = lax.axis_index("s") * NC + lax.axis_index("c")
        base = wid * b_per_w
        iota16 = lax.iota(jnp.int32, 16)

        pltpu.sync_copy(pu_h.at[pl.ds(base, b_per_w)], idx_u)
        pltpu.sync_copy(pu_p.at[pl.ds(base, b_per_w)], par_u)
        pltpu.sync_copy(pv_h.at[pl.ds(base, b_per_w)], idx_v)
        pltpu.sync_copy(pv_p.at[pl.ds(base, b_per_w)], par_v)
        nb = N_NEG * b_per_w
        pltpu.sync_copy(ng_h.at[pl.ds(wid * nb, nb)], idx_n)
        pltpu.sync_copy(ng_p.at[pl.ds(wid * nb, nb)], par_n)

        for c in range(n_chunk):
            co = c * CHUNK
            cps = [
                pltpu.async_copy(u2.at[idx_u.at[pl.ds(co, CHUNK)]], u_buf,
                                 sem),
                pltpu.async_copy(v2.at[idx_v.at[pl.ds(co, CHUNK)]], v_buf,
                                 sem),
            ]
            for n in range(N_NEG):
                cps.append(pltpu.async_copy(
                    v2.at[idx_n.at[pl.ds(n * b_per_w + co, CHUNK)]],
                    n_buf.at[pl.ds(n * CHUNK, CHUNK)], sem))
            for cp in cps:
                cp.wait()

            for g in range(CHUNK // 16):
                go = g * 16
                rows = go + iota16
                cu0 = par_u[pl.ds(co + go, 16)]
                cv0 = par_v[pl.ds(co + go, 16)]
                cn0 = [par_n[pl.ds(n * b_per_w + co + go, 16)]
                       for n in range(N_NEG)]
                rows_n = [rows + n * CHUNK for n in range(N_NEG)]

                def body(dd, accs):
                    ds16 = jnp.full((16,), dd, jnp.int32)
                    uvec = plsc.load_gather(u_buf, [rows, cu0 + ds16])
                    vvec = plsc.load_gather(v_buf, [rows, cv0 + ds16])
                    new = [accs[0] + uvec * vvec]
                    for n in range(N_NEG):
                        nv = plsc.load_gather(
                            n_buf, [rows_n[n], cn0[n] + ds16])
                        new.append(accs[1 + n] + nv * uvec)
                    return tuple(new)

                accs = lax.fori_loop(
                    0, d, body,
                    tuple(jnp.zeros((16,), jnp.float32) for _ in range(6)))
                for r in range(6):
                    acc_buf[r, pl.ds(co + go, 16)] = accs[r]

        for r in range(6):
            pltpu.sync_copy(acc_buf.at[r], out.at[pl.ds(r * batch + base,
                                                        b_per_w)])

    return k


def _tc_reduce(scores):
    def body(s_ref, o_ref):
        s = s_ref[...]
        rid = lax.broadcasted_iota(jnp.int32, s.shape, 0)
        valid = rid < 6
        sign = jnp.where(rid == 0, 1.0, -1.0)
        x = jnp.where(valid, s * sign, 0.0)
        vals = jax.nn.log_sigmoid(x)
        o_ref[0, 0] = -jnp.sum(jnp.where(valid, vals, 0.0))

    return pl.pallas_call(
        body,
        out_shape=jax.ShapeDtypeStruct((1, 1), jnp.float32),
        in_specs=[pl.BlockSpec(memory_space=pltpu.VMEM)],
        out_specs=pl.BlockSpec(memory_space=pltpu.SMEM),
    )(scores)


def kernel(pos_u, pos_v, neg_v, u_weight, v_weight):
    batch = pos_u.shape[0]
    vocab, d = u_weight.shape
    b_per_w = batch // NW
    n_chunk = b_per_w // CHUNK

    pos_u = pos_u.astype(jnp.int32)
    pos_v = pos_v.astype(jnp.int32)
    neg_t = (neg_v.astype(jnp.int32).T
             .reshape(N_NEG, NW, b_per_w).transpose(1, 0, 2).reshape(-1))

    pu_h, pu_p = pos_u >> 1, (pos_u & 1) * d
    pv_h, pv_p = pos_v >> 1, (pos_v & 1) * d
    ng_h, ng_p = neg_t >> 1, (neg_t & 1) * d

    u2 = u_weight.reshape(vocab // 2, 2 * d)
    v2 = v_weight.reshape(vocab // 2, 2 * d)

    flat = _sc_scores(b_per_w, n_chunk, d)(pu_h, pu_p, pv_h, pv_p,
                                           ng_h, ng_p, u2, v2)
    return _tc_reduce(flat.reshape(8, batch))[0, 0]


# speedup vs baseline: 1.5235x; 1.5235x over previous
import functools

import jax
import jax.numpy as jnp
from jax import lax
from jax.experimental import pallas as pl
from jax.experimental.pallas import tpu as pltpu
from jax.experimental.pallas import tpu_sc as plsc

NC = 2
NS = 16
NW = NC * NS
N_NEG = 5
CHUNK = 64


def _sc_scores(b_per_w, n_chunk, d):
    mesh = plsc.VectorSubcoreMesh(core_axis_name="c", subcore_axis_name="s")
    batch = b_per_w * NW
    w2 = 2 * d

    @functools.partial(
        pl.kernel,
        mesh=mesh,
        out_type=jax.ShapeDtypeStruct((8 * batch,), jnp.float32),
        compiler_params=pltpu.CompilerParams(needs_layout_passes=False,
                                             use_tc_tiling_on_sc=True),
        scratch_types=[
            pltpu.VMEM((b_per_w,), jnp.int32),
            pltpu.VMEM((b_per_w,), jnp.int32),
            pltpu.VMEM((b_per_w,), jnp.int32),
            pltpu.VMEM((b_per_w,), jnp.int32),
            pltpu.VMEM((N_NEG * b_per_w,), jnp.int32),
            pltpu.VMEM((N_NEG * b_per_w,), jnp.int32),
            pltpu.VMEM((CHUNK, w2), jnp.float32),
            pltpu.VMEM((CHUNK, w2), jnp.float32),
            pltpu.VMEM((N_NEG * CHUNK, w2), jnp.float32),
            pltpu.VMEM((6, b_per_w), jnp.float32),
            pltpu.SemaphoreType.DMA,
        ],
    )
    def k(pu_h, pu_p, pv_h, pv_p, ng_h, ng_p, u2, v2, out,
          idx_u, par_u, idx_v, par_v, idx_n, par_n,
          u_buf, v_buf, n_buf, acc_buf, sem):
        wid = lax.axis_index("s") * NC + lax.axis_index("c")
        base = wid * b_per_w
        iota16 = lax.iota(jnp.int32, 16)

        pltpu.sync_copy(pu_h.at[pl.ds(base, b_per_w)], idx_u)
        pltpu.sync_copy(pu_p.at[pl.ds(base, b_per_w)], par_u)
        pltpu.sync_copy(pv_h.at[pl.ds(base, b_per_w)], idx_v)
        pltpu.sync_copy(pv_p.at[pl.ds(base, b_per_w)], par_v)
        nb = N_NEG * b_per_w
        pltpu.sync_copy(ng_h.at[pl.ds(wid * nb, nb)], idx_n)
        pltpu.sync_copy(ng_p.at[pl.ds(wid * nb, nb)], par_n)

        for c in range(n_chunk):
            co = c * CHUNK
            cps = [
                pltpu.async_copy(u2.at[idx_u.at[pl.ds(co, CHUNK)]], u_buf,
                                 sem),
                pltpu.async_copy(v2.at[idx_v.at[pl.ds(co, CHUNK)]], v_buf,
                                 sem),
            ]
            for n in range(N_NEG):
                cps.append(pltpu.async_copy(
                    v2.at[idx_n.at[pl.ds(n * b_per_w + co, CHUNK)]],
                    n_buf.at[pl.ds(n * CHUNK, CHUNK)], sem))
            for cp in cps:
                cp.wait()

            for g in range(CHUNK // 16):
                go = g * 16
                rows = go + iota16
                cu0 = par_u[pl.ds(co + go, 16)]
                cv0 = par_v[pl.ds(co + go, 16)]
                cn0 = [par_n[pl.ds(n * b_per_w + co + go, 16)]
                       for n in range(N_NEG)]
                rows_n = [rows + n * CHUNK for n in range(N_NEG)]

                def body(dd, accs):
                    ds16 = jnp.full((16,), dd, jnp.int32)
                    uvec = plsc.load_gather(u_buf, [rows, cu0 + ds16])
                    vvec = plsc.load_gather(v_buf, [rows, cv0 + ds16])
                    new = [accs[0] + uvec * vvec]
                    for n in range(N_NEG):
                        nv = plsc.load_gather(
                            n_buf, [rows_n[n], cn0[n] + ds16])
                        new.append(accs[1 + n] + nv * uvec)
                    return tuple(new)

                accs = lax.fori_loop(
                    0, d, body,
                    tuple(jnp.zeros((16,), jnp.float32) for _ in range(6)))
                for r in range(6):
                    acc_buf[r, pl.ds(co + go, 16)] = accs[r]

        for r in range(6):
            pltpu.sync_copy(acc_buf.at[r], out.at[pl.ds(r * batch + base,
                                                        b_per_w)])

    return k


PAIR_BLK = 2048


def _pair_split(v):
    return PAIR_BLK * pl.cdiv(v // 2, PAIR_BLK)


def _tc_pair_rows(wt):
    d, v = wt.shape
    h = _pair_split(v)
    grid = h // PAIR_BLK

    def body(xa_ref, xb_ref, o_ref):
        o_ref[...] = jnp.concatenate(
            [xa_ref[...].T, xb_ref[...].T], axis=1)

    hb = h // PAIR_BLK
    vb_max = pl.cdiv(v, PAIR_BLK) - 1

    return pl.pallas_call(
        body,
        grid=(grid,),
        in_specs=[
            pl.BlockSpec((d, PAIR_BLK), lambda i: (0, i)),
            pl.BlockSpec(
                (d, PAIR_BLK),
                lambda i, hb=hb, vb_max=vb_max:
                    (0, jnp.minimum(i + hb, vb_max))),
        ],
        out_specs=pl.BlockSpec((PAIR_BLK, 2 * d), lambda i: (i, 0)),
        out_shape=jax.ShapeDtypeStruct((h, 2 * d), jnp.float32),
    )(wt, wt)


def _tc_reduce(scores):
    def body(s_ref, o_ref):
        s = s_ref[...]
        rid = lax.broadcasted_iota(jnp.int32, s.shape, 0)
        valid = rid < 6
        sign = jnp.where(rid == 0, 1.0, -1.0)
        x = jnp.where(valid, s * sign, 0.0)
        vals = jax.nn.log_sigmoid(x)
        o_ref[0, 0] = -jnp.sum(jnp.where(valid, vals, 0.0))

    return pl.pallas_call(
        body,
        out_shape=jax.ShapeDtypeStruct((1, 1), jnp.float32),
        in_specs=[pl.BlockSpec(memory_space=pltpu.VMEM)],
        out_specs=pl.BlockSpec(memory_space=pltpu.SMEM),
    )(scores)


def kernel(pos_u, pos_v, neg_v, u_weight, v_weight):
    batch = pos_u.shape[0]
    vocab, d = u_weight.shape
    b_per_w = batch // NW
    n_chunk = b_per_w // CHUNK

    pos_u = pos_u.astype(jnp.int32)
    pos_v = pos_v.astype(jnp.int32)
    neg_t = (neg_v.astype(jnp.int32).T
             .reshape(N_NEG, NW, b_per_w).transpose(1, 0, 2).reshape(-1))

    h = _pair_split(vocab)
    pu_h, pu_p = pos_u % h, (pos_u >= h).astype(jnp.int32) * d
    pv_h, pv_p = pos_v % h, (pos_v >= h).astype(jnp.int32) * d
    ng_h, ng_p = neg_t % h, (neg_t >= h).astype(jnp.int32) * d

    u2 = _tc_pair_rows(u_weight.T)
    v2 = _tc_pair_rows(v_weight.T)

    flat = _sc_scores(b_per_w, n_chunk, d)(pu_h, pu_p, pv_h, pv_p,
                                           ng_h, ng_p, u2, v2)
    return _tc_reduce(flat.reshape(8, batch))[0, 0]


# speedup vs baseline: 2.0084x; 1.3182x over previous
import functools

import jax
import jax.numpy as jnp
from jax import lax
from jax.experimental import pallas as pl
from jax.experimental.pallas import tpu as pltpu
from jax.experimental.pallas import tpu_sc as plsc

NC = 2
NS = 16
NW = NC * NS
N_NEG = 5
CHUNK = 64


def _sc_scores(b_per_w, n_chunk, d):
    mesh = plsc.VectorSubcoreMesh(core_axis_name="c", subcore_axis_name="s")
    batch = b_per_w * NW
    w2 = 2 * d

    @functools.partial(
        pl.kernel,
        mesh=mesh,
        out_type=jax.ShapeDtypeStruct((8 * batch,), jnp.float32),
        compiler_params=pltpu.CompilerParams(needs_layout_passes=False,
                                             use_tc_tiling_on_sc=True),
        scratch_types=[
            pltpu.VMEM((b_per_w,), jnp.int32),
            pltpu.VMEM((b_per_w,), jnp.int32),
            pltpu.VMEM((b_per_w,), jnp.int32),
            pltpu.VMEM((b_per_w,), jnp.int32),
            pltpu.VMEM((N_NEG * b_per_w,), jnp.int32),
            pltpu.VMEM((N_NEG * b_per_w,), jnp.int32),
            pltpu.VMEM((CHUNK, w2), jnp.float32),
            pltpu.VMEM((CHUNK, w2), jnp.float32),
            pltpu.VMEM((N_NEG * CHUNK, w2), jnp.float32),
            pltpu.VMEM((6, b_per_w), jnp.float32),
            pltpu.SemaphoreType.DMA,
        ],
    )
    def k(pu_h, pu_p, pv_h, pv_p, ng_h, ng_p, u2, v2, out,
          idx_u, par_u, idx_v, par_v, idx_n, par_n,
          u_buf, v_buf, n_buf, acc_buf, sem):
        wid = lax.axis_index("s") * NC + lax.axis_index("c")
        base = wid * b_per_w
        iota16 = lax.iota(jnp.int32, 16)

        pltpu.sync_copy(pu_h.at[pl.ds(base, b_per_w)], idx_u)
        pltpu.sync_copy(pu_p.at[pl.ds(base, b_per_w)], par_u)
        pltpu.sync_copy(pv_h.at[pl.ds(base, b_per_w)], idx_v)
        pltpu.sync_copy(pv_p.at[pl.ds(base, b_per_w)], par_v)
        nb = N_NEG * b_per_w
        pltpu.sync_copy(ng_h.at[pl.ds(wid * nb, nb)], idx_n)
        pltpu.sync_copy(ng_p.at[pl.ds(wid * nb, nb)], par_n)

        for c in range(n_chunk):
            co = c * CHUNK
            cps = [
                pltpu.async_copy(u2.at[idx_u.at[pl.ds(co, CHUNK)]], u_buf,
                                 sem),
                pltpu.async_copy(v2.at[idx_v.at[pl.ds(co, CHUNK)]], v_buf,
                                 sem),
            ]
            for n in range(N_NEG):
                cps.append(pltpu.async_copy(
                    v2.at[idx_n.at[pl.ds(n * b_per_w + co, CHUNK)]],
                    n_buf.at[pl.ds(n * CHUNK, CHUNK)], sem))
            for cp in cps:
                cp.wait()

            for g in range(CHUNK // 16):
                go = g * 16
                rows = go + iota16
                cu0 = par_u[pl.ds(co + go, 16)]
                cv0 = par_v[pl.ds(co + go, 16)]
                cn0 = [par_n[pl.ds(n * b_per_w + co + go, 16)]
                       for n in range(N_NEG)]
                rows_n = [rows + n * CHUNK for n in range(N_NEG)]

                def body(dd, accs):
                    ds16 = jnp.full((16,), dd, jnp.int32)
                    uvec = plsc.load_gather(u_buf, [rows, cu0 + ds16])
                    vvec = plsc.load_gather(v_buf, [rows, cv0 + ds16])
                    new = [accs[0] + uvec * vvec]
                    for n in range(N_NEG):
                        nv = plsc.load_gather(
                            n_buf, [rows_n[n], cn0[n] + ds16])
                        new.append(accs[1 + n] + nv * uvec)
                    return tuple(new)

                accs = lax.fori_loop(
                    0, d, body,
                    tuple(jnp.zeros((16,), jnp.float32) for _ in range(6)))
                for r in range(6):
                    acc_buf[r, pl.ds(co + go, 16)] = accs[r]

        for r in range(6):
            pltpu.sync_copy(acc_buf.at[r], out.at[pl.ds(r * batch + base,
                                                        b_per_w)])

    return k


PAIR_BLK = 8192


def _pair_split(v):
    return PAIR_BLK * pl.cdiv(v // 2, PAIR_BLK)


def _tc_pair_rows(wt):
    d, v = wt.shape
    h = _pair_split(v)
    grid = h // PAIR_BLK

    def body(xa_ref, xb_ref, o_ref):
        o_ref[...] = jnp.concatenate(
            [xa_ref[...].T, xb_ref[...].T], axis=1)

    hb = h // PAIR_BLK
    vb_max = pl.cdiv(v, PAIR_BLK) - 1

    return pl.pallas_call(
        body,
        grid=(grid,),
        in_specs=[
            pl.BlockSpec((d, PAIR_BLK), lambda i: (0, i)),
            pl.BlockSpec(
                (d, PAIR_BLK),
                lambda i, hb=hb, vb_max=vb_max:
                    (0, jnp.minimum(i + hb, vb_max))),
        ],
        out_specs=pl.BlockSpec((PAIR_BLK, 2 * d), lambda i: (i, 0)),
        out_shape=jax.ShapeDtypeStruct((h, 2 * d), jnp.float32),
    )(wt, wt)


def _tc_reduce(scores):
    def body(s_ref, o_ref):
        s = s_ref[...]
        rid = lax.broadcasted_iota(jnp.int32, s.shape, 0)
        valid = rid < 6
        sign = jnp.where(rid == 0, 1.0, -1.0)
        x = jnp.where(valid, s * sign, 0.0)
        vals = jax.nn.log_sigmoid(x)
        o_ref[0, 0] = -jnp.sum(jnp.where(valid, vals, 0.0))

    return pl.pallas_call(
        body,
        out_shape=jax.ShapeDtypeStruct((1, 1), jnp.float32),
        in_specs=[pl.BlockSpec(memory_space=pltpu.VMEM)],
        out_specs=pl.BlockSpec(memory_space=pltpu.SMEM),
    )(scores)


def kernel(pos_u, pos_v, neg_v, u_weight, v_weight):
    batch = pos_u.shape[0]
    vocab, d = u_weight.shape
    b_per_w = batch // NW
    n_chunk = b_per_w // CHUNK

    pos_u = pos_u.astype(jnp.int32)
    pos_v = pos_v.astype(jnp.int32)
    neg_t = (neg_v.astype(jnp.int32).T
             .reshape(N_NEG, NW, b_per_w).transpose(1, 0, 2).reshape(-1))

    h = _pair_split(vocab)
    pu_h, pu_p = pos_u % h, (pos_u >= h).astype(jnp.int32) * d
    pv_h, pv_p = pos_v % h, (pos_v >= h).astype(jnp.int32) * d
    ng_h, ng_p = neg_t % h, (neg_t >= h).astype(jnp.int32) * d

    u2 = _tc_pair_rows(u_weight.T)
    v2 = _tc_pair_rows(v_weight.T)

    flat = _sc_scores(b_per_w, n_chunk, d)(pu_h, pu_p, pv_h, pv_p,
                                           ng_h, ng_p, u2, v2)
    return _tc_reduce(flat.reshape(8, batch))[0, 0]


# speedup vs baseline: 2.1035x; 1.0473x over previous
import functools

import jax
import jax.numpy as jnp
from jax import lax
from jax.experimental import pallas as pl
from jax.experimental.pallas import tpu as pltpu
from jax.experimental.pallas import tpu_sc as plsc

NC = 2
NS = 16
NW = NC * NS
N_NEG = 5
CHUNK = 64


def _sc_scores(b_per_w, n_chunk, d):
    mesh = plsc.VectorSubcoreMesh(core_axis_name="c", subcore_axis_name="s")
    batch = b_per_w * NW
    w2 = 2 * d

    @functools.partial(
        pl.kernel,
        mesh=mesh,
        out_type=jax.ShapeDtypeStruct((8 * batch,), jnp.float32),
        compiler_params=pltpu.CompilerParams(needs_layout_passes=False,
                                             use_tc_tiling_on_sc=True),
        scratch_types=[
            pltpu.VMEM((b_per_w,), jnp.int32),
            pltpu.VMEM((b_per_w,), jnp.int32),
            pltpu.VMEM((b_per_w,), jnp.int32),
            pltpu.VMEM((b_per_w,), jnp.int32),
            pltpu.VMEM((N_NEG * b_per_w,), jnp.int32),
            pltpu.VMEM((N_NEG * b_per_w,), jnp.int32),
            pltpu.VMEM((CHUNK, w2), jnp.float32),
            pltpu.VMEM((CHUNK, w2), jnp.float32),
            pltpu.VMEM((N_NEG * CHUNK, w2), jnp.float32),
            pltpu.VMEM((6, b_per_w), jnp.float32),
            pltpu.SemaphoreType.DMA,
        ],
    )
    def k(pu_h, pu_p, pv_h, pv_p, ng_h, ng_p, u2, v2, out,
          idx_u, par_u, idx_v, par_v, idx_n, par_n,
          u_buf, v_buf, n_buf, acc_buf, sem):
        wid = lax.axis_index("s") * NC + lax.axis_index("c")
        base = wid * b_per_w
        iota16 = lax.iota(jnp.int32, 16)

        pltpu.sync_copy(pu_h.at[pl.ds(base, b_per_w)], idx_u)
        pltpu.sync_copy(pu_p.at[pl.ds(base, b_per_w)], par_u)
        pltpu.sync_copy(pv_h.at[pl.ds(base, b_per_w)], idx_v)
        pltpu.sync_copy(pv_p.at[pl.ds(base, b_per_w)], par_v)
        nb = N_NEG * b_per_w
        pltpu.sync_copy(ng_h.at[pl.ds(wid * nb, nb)], idx_n)
        pltpu.sync_copy(ng_p.at[pl.ds(wid * nb, nb)], par_n)

        for c in range(n_chunk):
            co = c * CHUNK
            cps = [
                pltpu.async_copy(u2.at[idx_u.at[pl.ds(co, CHUNK)]], u_buf,
                                 sem),
                pltpu.async_copy(v2.at[idx_v.at[pl.ds(co, CHUNK)]], v_buf,
                                 sem),
            ]
            for n in range(N_NEG):
                cps.append(pltpu.async_copy(
                    v2.at[idx_n.at[pl.ds(n * b_per_w + co, CHUNK)]],
                    n_buf.at[pl.ds(n * CHUNK, CHUNK)], sem))
            for cp in cps:
                cp.wait()

            for g in range(CHUNK // 16):
                go = g * 16
                rows = go + iota16
                cu0 = par_u[pl.ds(co + go, 16)]
                cv0 = par_v[pl.ds(co + go, 16)]
                cn0 = [par_n[pl.ds(n * b_per_w + co + go, 16)]
                       for n in range(N_NEG)]
                rows_n = [rows + n * CHUNK for n in range(N_NEG)]

                def body(dd, accs):
                    ds16 = jnp.full((16,), dd, jnp.int32)
                    uvec = plsc.load_gather(u_buf, [rows, cu0 + ds16])
                    vvec = plsc.load_gather(v_buf, [rows, cv0 + ds16])
                    new = [accs[0] + uvec * vvec]
                    for n in range(N_NEG):
                        nv = plsc.load_gather(
                            n_buf, [rows_n[n], cn0[n] + ds16])
                        new.append(accs[1 + n] + nv * uvec)
                    return tuple(new)

                accs = lax.fori_loop(
                    0, d, body,
                    tuple(jnp.zeros((16,), jnp.float32) for _ in range(6)))
                for r in range(6):
                    acc_buf[r, pl.ds(co + go, 16)] = accs[r]

        for r in range(6):
            pltpu.sync_copy(acc_buf.at[r], out.at[pl.ds(r * batch + base,
                                                        b_per_w)])

    return k


PAIR_BLK = 16384


def _pair_split(v):
    return PAIR_BLK * pl.cdiv(v // 2, PAIR_BLK)


def _tc_pair_rows(wt):
    d, v = wt.shape
    h = _pair_split(v)
    grid = h // PAIR_BLK

    def body(xa_ref, xb_ref, o_ref):
        o_ref[...] = jnp.concatenate(
            [xa_ref[...].T, xb_ref[...].T], axis=1)

    hb = h // PAIR_BLK
    vb_max = pl.cdiv(v, PAIR_BLK) - 1

    return pl.pallas_call(
        body,
        grid=(grid,),
        in_specs=[
            pl.BlockSpec((d, PAIR_BLK), lambda i: (0, i)),
            pl.BlockSpec(
                (d, PAIR_BLK),
                lambda i, hb=hb, vb_max=vb_max:
                    (0, jnp.minimum(i + hb, vb_max))),
        ],
        out_specs=pl.BlockSpec((PAIR_BLK, 2 * d), lambda i: (i, 0)),
        out_shape=jax.ShapeDtypeStruct((h, 2 * d), jnp.float32),
    )(wt, wt)


def _tc_reduce(scores):
    def body(s_ref, o_ref):
        s = s_ref[...]
        rid = lax.broadcasted_iota(jnp.int32, s.shape, 0)
        valid = rid < 6
        sign = jnp.where(rid == 0, 1.0, -1.0)
        x = jnp.where(valid, s * sign, 0.0)
        vals = jax.nn.log_sigmoid(x)
        o_ref[0, 0] = -jnp.sum(jnp.where(valid, vals, 0.0))

    return pl.pallas_call(
        body,
        out_shape=jax.ShapeDtypeStruct((1, 1), jnp.float32),
        in_specs=[pl.BlockSpec(memory_space=pltpu.VMEM)],
        out_specs=pl.BlockSpec(memory_space=pltpu.SMEM),
    )(scores)


def kernel(pos_u, pos_v, neg_v, u_weight, v_weight):
    batch = pos_u.shape[0]
    vocab, d = u_weight.shape
    b_per_w = batch // NW
    n_chunk = b_per_w // CHUNK

    pos_u = pos_u.astype(jnp.int32)
    pos_v = pos_v.astype(jnp.int32)
    neg_t = (neg_v.astype(jnp.int32).T
             .reshape(N_NEG, NW, b_per_w).transpose(1, 0, 2).reshape(-1))

    h = _pair_split(vocab)
    pu_h, pu_p = pos_u % h, (pos_u >= h).astype(jnp.int32) * d
    pv_h, pv_p = pos_v % h, (pos_v >= h).astype(jnp.int32) * d
    ng_h, ng_p = neg_t % h, (neg_t >= h).astype(jnp.int32) * d

    u2 = _tc_pair_rows(u_weight.T)
    v2 = _tc_pair_rows(v_weight.T)

    flat = _sc_scores(b_per_w, n_chunk, d)(pu_h, pu_p, pv_h, pv_p,
                                           ng_h, ng_p, u2, v2)
    return _tc_reduce(flat.reshape(8, batch))[0, 0]


# speedup vs baseline: 2.2203x; 1.0556x over previous
import functools

import jax
import jax.numpy as jnp
from jax import lax
from jax.experimental import pallas as pl
from jax.experimental.pallas import tpu as pltpu
from jax.experimental.pallas import tpu_sc as plsc

NC = 2
NS = 16
NW = NC * NS
N_NEG = 5
CHUNK = 64


def _sc_scores(b_per_w, n_chunk, d):
    mesh = plsc.VectorSubcoreMesh(core_axis_name="c", subcore_axis_name="s")
    batch = b_per_w * NW
    w2 = 2 * d

    @functools.partial(
        pl.kernel,
        mesh=mesh,
        out_type=jax.ShapeDtypeStruct((8 * batch,), jnp.float32),
        compiler_params=pltpu.CompilerParams(needs_layout_passes=False,
                                             use_tc_tiling_on_sc=True),
        scratch_types=[
            pltpu.VMEM((b_per_w,), jnp.int32),
            pltpu.VMEM((b_per_w,), jnp.int32),
            pltpu.VMEM((b_per_w,), jnp.int32),
            pltpu.VMEM((b_per_w,), jnp.int32),
            pltpu.VMEM((N_NEG * b_per_w,), jnp.int32),
            pltpu.VMEM((N_NEG * b_per_w,), jnp.int32),
            pltpu.VMEM((CHUNK, w2), jnp.float32),
            pltpu.VMEM((CHUNK, w2), jnp.float32),
            pltpu.VMEM((N_NEG * CHUNK, w2), jnp.float32),
            pltpu.VMEM((CHUNK, w2), jnp.float32),
            pltpu.VMEM((CHUNK, w2), jnp.float32),
            pltpu.VMEM((N_NEG * CHUNK, w2), jnp.float32),
            pltpu.VMEM((6, b_per_w), jnp.float32),
            pltpu.SemaphoreType.DMA,
            pltpu.SemaphoreType.DMA,
        ],
    )
    def k(pu_h, pu_p, pv_h, pv_p, ng_h, ng_p, u2, v2, out,
          idx_u, par_u, idx_v, par_v, idx_n, par_n,
          u_buf_a, v_buf_a, n_buf_a, u_buf_b, v_buf_b, n_buf_b,
          acc_buf, sem_a, sem_b):
        wid = lax.axis_index("s") * NC + lax.axis_index("c")
        base = wid * b_per_w
        iota16 = lax.iota(jnp.int32, 16)

        pltpu.sync_copy(pu_h.at[pl.ds(base, b_per_w)], idx_u)
        pltpu.sync_copy(pu_p.at[pl.ds(base, b_per_w)], par_u)
        pltpu.sync_copy(pv_h.at[pl.ds(base, b_per_w)], idx_v)
        pltpu.sync_copy(pv_p.at[pl.ds(base, b_per_w)], par_v)
        nb = N_NEG * b_per_w
        pltpu.sync_copy(ng_h.at[pl.ds(wid * nb, nb)], idx_n)
        pltpu.sync_copy(ng_p.at[pl.ds(wid * nb, nb)], par_n)

        bufs = [(u_buf_a, v_buf_a, n_buf_a, sem_a),
                (u_buf_b, v_buf_b, n_buf_b, sem_b)]

        def fire(c):
            ub, vb, nbf, sem = bufs[c % 2]
            co = c * CHUNK
            cps = [
                pltpu.async_copy(u2.at[idx_u.at[pl.ds(co, CHUNK)]], ub, sem),
                pltpu.async_copy(v2.at[idx_v.at[pl.ds(co, CHUNK)]], vb, sem),
            ]
            for n in range(N_NEG):
                cps.append(pltpu.async_copy(
                    v2.at[idx_n.at[pl.ds(n * b_per_w + co, CHUNK)]],
                    nbf.at[pl.ds(n * CHUNK, CHUNK)], sem))
            return cps

        cps = fire(0)
        for c in range(n_chunk):
            co = c * CHUNK
            u_buf, v_buf, n_buf, _ = bufs[c % 2]
            cps_next = fire(c + 1) if c + 1 < n_chunk else []
            for cp in cps:
                cp.wait()
            cps = cps_next

            for g in range(CHUNK // 16):
                go = g * 16
                rows = go + iota16
                cu0 = par_u[pl.ds(co + go, 16)]
                cv0 = par_v[pl.ds(co + go, 16)]
                cn0 = [par_n[pl.ds(n * b_per_w + co + go, 16)]
                       for n in range(N_NEG)]
                rows_n = [rows + n * CHUNK for n in range(N_NEG)]

                def body(dd, accs):
                    ds16 = jnp.full((16,), dd, jnp.int32)
                    uvec = plsc.load_gather(u_buf, [rows, cu0 + ds16])
                    vvec = plsc.load_gather(v_buf, [rows, cv0 + ds16])
                    new = [accs[0] + uvec * vvec]
                    for n in range(N_NEG):
                        nv = plsc.load_gather(
                            n_buf, [rows_n[n], cn0[n] + ds16])
                        new.append(accs[1 + n] + nv * uvec)
                    return tuple(new)

                accs = lax.fori_loop(
                    0, d, body,
                    tuple(jnp.zeros((16,), jnp.float32) for _ in range(6)),
                    unroll=4)
                for r in range(6):
                    acc_buf[r, pl.ds(co + go, 16)] = accs[r]

        for r in range(6):
            pltpu.sync_copy(acc_buf.at[r], out.at[pl.ds(r * batch + base,
                                                        b_per_w)])

    return k


PAIR_BLK = 16384


def _pair_split(v):
    return PAIR_BLK * pl.cdiv(v // 2, PAIR_BLK)


def _tc_pair_rows(wt):
    d, v = wt.shape
    h = _pair_split(v)
    grid = h // PAIR_BLK

    def body(xa_ref, xb_ref, o_ref):
        o_ref[...] = jnp.concatenate(
            [xa_ref[...].T, xb_ref[...].T], axis=1)

    hb = h // PAIR_BLK
    vb_max = pl.cdiv(v, PAIR_BLK) - 1

    return pl.pallas_call(
        body,
        grid=(grid,),
        in_specs=[
            pl.BlockSpec((d, PAIR_BLK), lambda i: (0, i)),
            pl.BlockSpec(
                (d, PAIR_BLK),
                lambda i, hb=hb, vb_max=vb_max:
                    (0, jnp.minimum(i + hb, vb_max))),
        ],
        out_specs=pl.BlockSpec((PAIR_BLK, 2 * d), lambda i: (i, 0)),
        out_shape=jax.ShapeDtypeStruct((h, 2 * d), jnp.float32),
    )(wt, wt)


def _tc_reduce(scores):
    def body(s_ref, o_ref):
        s = s_ref[...]
        rid = lax.broadcasted_iota(jnp.int32, s.shape, 0)
        valid = rid < 6
        sign = jnp.where(rid == 0, 1.0, -1.0)
        x = jnp.where(valid, s * sign, 0.0)
        vals = jax.nn.log_sigmoid(x)
        o_ref[0, 0] = -jnp.sum(jnp.where(valid, vals, 0.0))

    return pl.pallas_call(
        body,
        out_shape=jax.ShapeDtypeStruct((1, 1), jnp.float32),
        in_specs=[pl.BlockSpec(memory_space=pltpu.VMEM)],
        out_specs=pl.BlockSpec(memory_space=pltpu.SMEM),
    )(scores)


def kernel(pos_u, pos_v, neg_v, u_weight, v_weight):
    batch = pos_u.shape[0]
    vocab, d = u_weight.shape
    b_per_w = batch // NW
    n_chunk = b_per_w // CHUNK

    pos_u = pos_u.astype(jnp.int32)
    pos_v = pos_v.astype(jnp.int32)
    neg_t = (neg_v.astype(jnp.int32).T
             .reshape(N_NEG, NW, b_per_w).transpose(1, 0, 2).reshape(-1))

    h = _pair_split(vocab)
    pu_h, pu_p = pos_u % h, (pos_u >= h).astype(jnp.int32) * d
    pv_h, pv_p = pos_v % h, (pos_v >= h).astype(jnp.int32) * d
    ng_h, ng_p = neg_t % h, (neg_t >= h).astype(jnp.int32) * d

    u2 = _tc_pair_rows(u_weight.T)
    v2 = _tc_pair_rows(v_weight.T)

    flat = _sc_scores(b_per_w, n_chunk, d)(pu_h, pu_p, pv_h, pv_p,
                                           ng_h, ng_p, u2, v2)
    return _tc_reduce(flat.reshape(8, batch))[0, 0]


# speedup vs baseline: 2.3236x; 1.0465x over previous
import functools

import jax
import jax.numpy as jnp
from jax import lax
from jax.experimental import pallas as pl
from jax.experimental.pallas import tpu as pltpu
from jax.experimental.pallas import tpu_sc as plsc

NC = 2
NS = 16
NW = NC * NS
N_NEG = 5
CHUNK = 64
MERGE_BLK = 16384


def _sc_scores(b_per_w, n_chunk, d):
    mesh = plsc.VectorSubcoreMesh(core_axis_name="c", subcore_axis_name="s")
    batch = b_per_w * NW
    w2 = 2 * d

    @functools.partial(
        pl.kernel,
        mesh=mesh,
        out_type=jax.ShapeDtypeStruct((8 * batch,), jnp.float32),
        compiler_params=pltpu.CompilerParams(needs_layout_passes=False,
                                             use_tc_tiling_on_sc=True),
        scratch_types=[
            pltpu.VMEM((b_per_w,), jnp.int32),
            pltpu.VMEM((b_per_w,), jnp.int32),
            pltpu.VMEM((N_NEG * b_per_w,), jnp.int32),
            pltpu.VMEM((CHUNK, w2), jnp.float32),
            pltpu.VMEM((CHUNK, w2), jnp.float32),
            pltpu.VMEM((N_NEG * CHUNK, w2), jnp.float32),
            pltpu.VMEM((CHUNK, w2), jnp.float32),
            pltpu.VMEM((CHUNK, w2), jnp.float32),
            pltpu.VMEM((N_NEG * CHUNK, w2), jnp.float32),
            pltpu.VMEM((6, b_per_w), jnp.float32),
            pltpu.SemaphoreType.DMA,
            pltpu.SemaphoreType.DMA,
        ],
    )
    def k(pu, pv, ng, uv2, out,
          idx_u, idx_v, idx_n,
          u_buf_a, v_buf_a, n_buf_a, u_buf_b, v_buf_b, n_buf_b,
          acc_buf, sem_a, sem_b):
        wid = lax.axis_index("s") * NC + lax.axis_index("c")
        base = wid * b_per_w
        iota16 = lax.iota(jnp.int32, 16)

        pltpu.sync_copy(pu.at[pl.ds(base, b_per_w)], idx_u)
        pltpu.sync_copy(pv.at[pl.ds(base, b_per_w)], idx_v)
        nb = N_NEG * b_per_w
        pltpu.sync_copy(ng.at[pl.ds(wid * nb, nb)], idx_n)

        bufs = [(u_buf_a, v_buf_a, n_buf_a, sem_a),
                (u_buf_b, v_buf_b, n_buf_b, sem_b)]

        def fire(c):
            ub, vb, nbf, sem = bufs[c % 2]
            co = c * CHUNK
            cps = [
                pltpu.async_copy(uv2.at[idx_u.at[pl.ds(co, CHUNK)]], ub, sem),
                pltpu.async_copy(uv2.at[idx_v.at[pl.ds(co, CHUNK)]], vb, sem),
            ]
            for n in range(N_NEG):
                cps.append(pltpu.async_copy(
                    uv2.at[idx_n.at[pl.ds(n * b_per_w + co, CHUNK)]],
                    nbf.at[pl.ds(n * CHUNK, CHUNK)], sem))
            return cps

        cps = fire(0)
        for c in range(n_chunk):
            co = c * CHUNK
            u_buf, v_buf, n_buf, _ = bufs[c % 2]
            cps_next = fire(c + 1) if c + 1 < n_chunk else []
            for cp in cps:
                cp.wait()
            cps = cps_next

            for g in range(CHUNK // 16):
                go = g * 16
                rows = go + iota16
                rows_n = [rows + n * CHUNK for n in range(N_NEG)]

                def body(dd, accs):
                    cu = jnp.full((16,), dd, jnp.int32)
                    cv = cu + d
                    uvec = plsc.load_gather(u_buf, [rows, cu])
                    vvec = plsc.load_gather(v_buf, [rows, cv])
                    new = [accs[0] + uvec * vvec]
                    for n in range(N_NEG):
                        nv = plsc.load_gather(n_buf, [rows_n[n], cv])
                        new.append(accs[1 + n] + nv * uvec)
                    return tuple(new)

                accs = lax.fori_loop(
                    0, d, body,
                    tuple(jnp.zeros((16,), jnp.float32) for _ in range(6)),
                    unroll=4)
                for r in range(6):
                    acc_buf[r, pl.ds(co + go, 16)] = accs[r]

        for r in range(6):
            pltpu.sync_copy(acc_buf.at[r], out.at[pl.ds(r * batch + base,
                                                        b_per_w)])

    return k


def _tc_merge_tables(ut, vt):
    d, v = ut.shape
    grid = pl.cdiv(v, MERGE_BLK)

    def body(ua_ref, va_ref, o_ref):
        o_ref[...] = jnp.concatenate(
            [ua_ref[...].T, va_ref[...].T], axis=1)

    return pl.pallas_call(
        body,
        grid=(grid,),
        in_specs=[
            pl.BlockSpec((d, MERGE_BLK), lambda i: (0, i)),
            pl.BlockSpec((d, MERGE_BLK), lambda i: (0, i)),
        ],
        out_specs=pl.BlockSpec((MERGE_BLK, 2 * d), lambda i: (i, 0)),
        out_shape=jax.ShapeDtypeStruct((v, 2 * d), jnp.float32),
    )(ut, vt)


def _tc_reduce(scores):
    def body(s_ref, o_ref):
        s = s_ref[...]
        rid = lax.broadcasted_iota(jnp.int32, s.shape, 0)
        valid = rid < 6
        sign = jnp.where(rid == 0, 1.0, -1.0)
        x = jnp.where(valid, s * sign, 0.0)
        vals = jax.nn.log_sigmoid(x)
        o_ref[0, 0] = -jnp.sum(jnp.where(valid, vals, 0.0))

    return pl.pallas_call(
        body,
        out_shape=jax.ShapeDtypeStruct((1, 1), jnp.float32),
        in_specs=[pl.BlockSpec(memory_space=pltpu.VMEM)],
        out_specs=pl.BlockSpec(memory_space=pltpu.SMEM),
    )(scores)


def kernel(pos_u, pos_v, neg_v, u_weight, v_weight):
    batch = pos_u.shape[0]
    vocab, d = u_weight.shape
    b_per_w = batch // NW
    n_chunk = b_per_w // CHUNK

    pos_u = pos_u.astype(jnp.int32)
    pos_v = pos_v.astype(jnp.int32)
    neg_t = (neg_v.astype(jnp.int32).T
             .reshape(N_NEG, NW, b_per_w).transpose(1, 0, 2).reshape(-1))

    uv2 = _tc_merge_tables(u_weight.T, v_weight.T)

    flat = _sc_scores(b_per_w, n_chunk, d)(pos_u, pos_v, neg_t, uv2)
    return _tc_reduce(flat.reshape(8, batch))[0, 0]
